# Initial kernel scaffold; baseline (speedup 1.0000x reference)
import jax
import jax.numpy as jnp
from jax.experimental import pallas as pl


def kernel(q, a, edge_index, embed, in_W0, in_b0, in_W1, in_b1, in_W2, in_b2, in_W3, in_b3, msg_W0, msg_b0, msg_W1, msg_b1, msg_W2, msg_b2, msg_W3, msg_b3, W_ih, W_hh, out_W, out_b):
    raise NotImplementedError("write your pallas kernel here")


# speedup vs baseline: 1.4710x; 1.4710x over previous
import jax
import jax.numpy as jnp
from jax import lax
from jax.experimental import pallas as pl
from jax.experimental.pallas import tpu as pltpu
from jax.experimental.pallas import tpu_sc as plsc

N = 10000
H = 84
E = 160000
STEPS = 4

NC = 2
NS = 16
NW = NC * NS
CHUNK = 128
CPW = 40
EPW = CHUNK * CPW
E_PAD = NW * EPW
NROWS = NW * CPW
RPS = 632
NM = RPS * NS

BN = 2000
BE = 2048
HP = 128


def _full(shape):
    return pl.BlockSpec(shape, lambda *_: tuple(0 for _ in shape))


def _input_body(q_ref, emb_ref, w0, b0, w1, b1, w2, b2, w3, b3,
                w0a, w0b, b0m, x_ref, p_ref, qq_ref):
    t = emb_ref[...]
    t = jnp.maximum(jnp.dot(t, w0[...], preferred_element_type=jnp.float32) + b0[...], 0.0)
    t = jnp.maximum(jnp.dot(t, w1[...], preferred_element_type=jnp.float32) + b1[...], 0.0)
    t = jnp.maximum(jnp.dot(t, w2[...], preferred_element_type=jnp.float32) + b2[...], 0.0)
    table = jnp.dot(t, w3[...], preferred_element_type=jnp.float32) + b3[...]
    io = lax.broadcasted_iota(jnp.int32, (BN, 9), 1)
    oh = (io == q_ref[...]).astype(jnp.float32)
    x = jnp.dot(oh, table, preferred_element_type=jnp.float32)
    x_ref[...] = x
    p_ref[...] = jnp.dot(x, w0a[...], preferred_element_type=jnp.float32) + b0m[...]
    qq_ref[...] = jnp.dot(x, w0b[...], preferred_element_type=jnp.float32)


def _input_mlp(q2, embed, iw, ib, w0a, w0b, b0m):
    f32 = jnp.float32
    return pl.pallas_call(
        _input_body,
        grid=(N // BN,),
        in_specs=[
            pl.BlockSpec((BN, 1), lambda i: (i, 0)),
            _full((9, 16)),
            _full((16, H)), _full((1, H)),
            _full((H, H)), _full((1, H)),
            _full((H, H)), _full((1, H)),
            _full((H, H)), _full((1, H)),
            _full((H, HP)), _full((H, HP)), _full((1, HP)),
        ],
        out_specs=[
            pl.BlockSpec((BN, H), lambda i: (i, 0)),
            pl.BlockSpec((BN, HP), lambda i: (i, 0)),
            pl.BlockSpec((BN, HP), lambda i: (i, 0)),
        ],
        out_shape=[jax.ShapeDtypeStruct((N, H), f32),
                   jax.ShapeDtypeStruct((N, HP), f32),
                   jax.ShapeDtypeStruct((N, HP), f32)],
    )(q2, embed, iw[0], ib[0], iw[1], ib[1], iw[2], ib[2], iw[3], ib[3],
      w0a, w0b, b0m)


def _edge_body(a_ref, b_ref, w1, b1, w2, b2, w3, b3, e_ref):
    t = jnp.maximum(a_ref[...] + b_ref[...], 0.0)
    t = jnp.maximum(jnp.dot(t, w1[...], preferred_element_type=jnp.float32) + b1[...], 0.0)
    t = jnp.maximum(jnp.dot(t, w2[...], preferred_element_type=jnp.float32) + b2[...], 0.0)
    e_ref[...] = jnp.dot(t, w3[...], preferred_element_type=jnp.float32) + b3[...]


def _edge_mlp(ea, eb, w1p, b1, w2, b2, w3p, b3p):
    return pl.pallas_call(
        _edge_body,
        grid=(E_PAD // BE,),
        in_specs=[
            pl.BlockSpec((BE, HP), lambda i: (i, 0)),
            pl.BlockSpec((BE, HP), lambda i: (i, 0)),
            _full((HP, H)), _full((1, H)),
            _full((H, H)), _full((1, H)),
            _full((H, HP)), _full((1, HP)),
        ],
        out_specs=pl.BlockSpec((BE, HP), lambda i: (i, 0)),
        out_shape=jax.ShapeDtypeStruct((E_PAD, HP), jnp.float32),
    )(ea, eb, w1p, b1, w2, b2, w3p, b3p)


def _sigmoid(x):
    return 1.0 / (1.0 + jnp.exp(-x))


def _lstm_body(x_ref, m0_ref, m1_ref, h_ref, c_ref,
               wxi, wmi, whi, wxf, wmf, whf, wxg, wmg, whg, wxo, wmo, who,
               w0a, w0b, b0m,
               h_out, c_out, p_out, q_out):
    x = x_ref[...]
    m = m0_ref[0] + m1_ref[0]
    h = h_ref[...]

    def gate(wx, wm, wh):
        return (jnp.dot(x, wx[...], preferred_element_type=jnp.float32)
                + jnp.dot(m, wm[...], preferred_element_type=jnp.float32)
                + jnp.dot(h, wh[...], preferred_element_type=jnp.float32))

    i_g = _sigmoid(gate(wxi, wmi, whi))
    f_g = _sigmoid(gate(wxf, wmf, whf))
    g_g = jnp.tanh(gate(wxg, wmg, whg))
    o_g = _sigmoid(gate(wxo, wmo, who))
    c_new = f_g * c_ref[...] + i_g * g_g
    h_new = o_g * jnp.tanh(c_new)
    h_out[...] = h_new
    c_out[...] = c_new
    p_out[...] = jnp.dot(h_new, w0a[...], preferred_element_type=jnp.float32) + b0m[...]
    q_out[...] = jnp.dot(h_new, w0b[...], preferred_element_type=jnp.float32)


def _lstm(x, mp, h, c, gw, w0a, w0b, b0m):
    f32 = jnp.float32
    return pl.pallas_call(
        _lstm_body,
        grid=(N // BN,),
        in_specs=[
            pl.BlockSpec((BN, H), lambda i: (i, 0)),
            pl.BlockSpec((1, BN, HP), lambda i: (0, i, 0)),
            pl.BlockSpec((1, BN, HP), lambda i: (1, i, 0)),
            pl.BlockSpec((BN, H), lambda i: (i, 0)),
            pl.BlockSpec((BN, H), lambda i: (i, 0)),
        ] + [_full((H, H)), _full((HP, H)), _full((H, H))] * 4
          + [_full((H, HP)), _full((H, HP)), _full((1, HP))],
        out_specs=[pl.BlockSpec((BN, H), lambda i: (i, 0))] * 2
          + [pl.BlockSpec((BN, HP), lambda i: (i, 0))] * 2,
        out_shape=[jax.ShapeDtypeStruct((N, H), f32)] * 2
          + [jax.ShapeDtypeStruct((N, HP), f32)] * 2,
    )(x, mp, mp, h, c, *gw, w0a, w0b, b0m)


def _readout_body(hs_ref, a_ref, w_ref, b_ref, preds_ref, loss_ref):
    s = pl.program_id(0)
    h = hs_ref[0]
    l = jnp.dot(h, w_ref[...], preferred_element_type=jnp.float32) + b_ref[...]
    mx = jnp.max(l, axis=1, keepdims=True)
    io = lax.broadcasted_iota(jnp.int32, (N, 9), 1)
    pred = jnp.min(jnp.where(l == mx, io, 9), axis=1)
    preds_ref[...] = pred.reshape(1, 1, N)
    lse = mx[:, 0] + jnp.log(jnp.sum(jnp.exp(l - mx), axis=1))
    l_lab = jnp.sum(jnp.where(io == a_ref[...], l, 0.0), axis=1)
    part = jnp.sum(lse - l_lab).reshape(1, 1)

    @pl.when(s == 0)
    def _():
        loss_ref[...] = jnp.zeros((1, 1), jnp.float32)

    loss_ref[...] += part

    @pl.when(s == STEPS - 1)
    def _():
        loss_ref[...] = loss_ref[...] / float(STEPS * N)


def _readout(hs, a2, out_w, out_b2):
    return pl.pallas_call(
        _readout_body,
        grid=(STEPS,),
        in_specs=[
            pl.BlockSpec((1, N, H), lambda s: (s, 0, 0)),
            pl.BlockSpec((N, 1), lambda s: (0, 0)),
            _full((H, 9)),
            _full((1, 9)),
        ],
        out_specs=[
            pl.BlockSpec((1, 1, N), lambda s: (s, 0, 0)),
            pl.BlockSpec((1, 1), lambda s: (0, 0)),
        ],
        out_shape=[
            jax.ShapeDtypeStruct((STEPS, 1, N), jnp.int32),
            jax.ShapeDtypeStruct((1, 1), jnp.float32),
        ],
    )(hs, a2, out_w, out_b2)


def _sc_gather_body(p_hbm, q_hbm, src_hbm, dst_hbm, ea_hbm, eb_hbm,
                    idx_s, idx_d, buf_a, buf_b, sem):
    wid = lax.axis_index("s") * NC + lax.axis_index("c")
    row0 = wid * CPW
    pltpu.sync_copy(src_hbm.at[pl.ds(row0, CPW)], idx_s)
    pltpu.sync_copy(dst_hbm.at[pl.ds(row0, CPW)], idx_d)

    def chunk(j, carry):
        off = (row0 + j) * CHUNK
        pltpu.async_copy(p_hbm.at[idx_s.at[j]], buf_a, sem).wait()
        pltpu.sync_copy(buf_a, ea_hbm.at[pl.ds(off, CHUNK)])
        pltpu.async_copy(q_hbm.at[idx_d.at[j]], buf_b, sem).wait()
        pltpu.sync_copy(buf_b, eb_hbm.at[pl.ds(off, CHUNK)])
        return carry

    lax.fori_loop(0, CPW, chunk, 0)


def _sc_gather(p, q, src_r, dst_r):
    f32 = jnp.float32
    return pl.kernel(
        _sc_gather_body,
        [jax.ShapeDtypeStruct((E_PAD, HP), f32)] * 2,
        mesh=plsc.VectorSubcoreMesh(core_axis_name="c", subcore_axis_name="s"),
        scratch_types=[
            pltpu.VMEM((CPW, CHUNK), jnp.int32),
            pltpu.VMEM((CPW, CHUNK), jnp.int32),
            pltpu.VMEM((CHUNK, HP), f32),
            pltpu.VMEM((CHUNK, HP), f32),
            pltpu.SemaphoreType.DMA,
        ],
    )(p, q, src_r, dst_r)


def _sc_scatter_body(e_hbm, dst_hbm, z_hbm, out_hbm, msh, idx_d, buf, sem):
    cid = lax.axis_index("c")
    sid = lax.axis_index("s")
    wid = sid * NC + cid
    row0 = wid * CPW
    pltpu.sync_copy(z_hbm.at[pl.ds(sid * RPS, RPS)], msh.at[pl.ds(sid * RPS, RPS)])
    pltpu.sync_copy(dst_hbm.at[pl.ds(row0, CPW)], idx_d)
    plsc.subcore_barrier()

    def chunk(j, carry):
        off = (row0 + j) * CHUNK
        pltpu.sync_copy(e_hbm.at[pl.ds(off, CHUNK)], buf)
        pltpu.sync_copy(buf, msh.at[idx_d.at[j]], add=True)
        return carry

    lax.fori_loop(0, CPW, chunk, 0)
    plsc.subcore_barrier()
    pltpu.sync_copy(msh.at[pl.ds(sid * RPS, RPS)],
                    out_hbm.at[cid, pl.ds(sid * RPS, RPS)])


def _sc_scatter(e, dst_r, zrows):
    f32 = jnp.float32
    return pl.kernel(
        _sc_scatter_body,
        jax.ShapeDtypeStruct((NC, NM, HP), f32),
        mesh=plsc.VectorSubcoreMesh(core_axis_name="c", subcore_axis_name="s"),
        scratch_types=[
            pltpu.VMEM_SHARED((NM, HP), f32),
            pltpu.VMEM((CPW, CHUNK), jnp.int32),
            pltpu.VMEM((CHUNK, HP), f32),
            pltpu.SemaphoreType.DMA,
        ],
    )(e, dst_r, zrows)


def kernel(q, a, edge_index, embed, in_W0, in_b0, in_W1, in_b1, in_W2, in_b2,
           in_W3, in_b3, msg_W0, msg_b0, msg_W1, msg_b1, msg_W2, msg_b2,
           msg_W3, msg_b3, W_ih, W_hh, out_W, out_b):
    f32 = jnp.float32
    i32 = jnp.int32
    q2 = q.astype(i32).reshape(N, 1)
    a2 = a.astype(i32).reshape(N, 1)
    ei = edge_index.astype(i32)
    pad = E_PAD - E
    src_r = jnp.concatenate([ei[0], jnp.zeros((pad,), i32)]).reshape(NROWS, CHUNK)
    dstg_r = jnp.concatenate([ei[1], jnp.zeros((pad,), i32)]).reshape(NROWS, CHUNK)
    dsts_r = jnp.concatenate([ei[1], jnp.full((pad,), N, i32)]).reshape(NROWS, CHUNK)
    zrows = jnp.zeros((NM, HP), f32)

    def padc(w):
        return jnp.pad(w, ((0, 0), (0, HP - w.shape[1])))

    def padr(w):
        return jnp.pad(w, ((0, HP - w.shape[0]), (0, 0)))

    iw = [in_W0, in_W1, in_W2, in_W3]
    ib = [b.reshape(1, H) for b in (in_b0, in_b1, in_b2, in_b3)]
    w0a = padc(msg_W0[:H])
    w0b = padc(msg_W0[H:])
    b0m = padc(msg_b0.reshape(1, H))
    w1p = padr(msg_W1)
    b1m = msg_b1.reshape(1, H)
    b2m = msg_b2.reshape(1, H)
    w3p = padc(msg_W3)
    b3p = padc(msg_b3.reshape(1, H))

    gw = []
    for k in range(4):
        gw.append(W_ih[k * H:(k + 1) * H, :H].T)
        gw.append(padr(W_ih[k * H:(k + 1) * H, H:].T))
        gw.append(W_hh[k * H:(k + 1) * H, :].T)

    x, p, qq = _input_mlp(q2, embed, iw, ib, w0a, w0b, b0m)
    h = x
    c = jnp.zeros((N, H), f32)
    hs_list = []
    for _ in range(STEPS):
        ea, eb = _sc_gather(p, qq, src_r, dstg_r)
        e = _edge_mlp(ea, eb, w1p, b1m, msg_W2, b2m, w3p, b3p)
        mp = _sc_scatter(e, dsts_r, zrows)
        h, c, p, qq = _lstm(x, mp, h, c, gw, w0a, w0b, b0m)
        hs_list.append(h)
    hs = jnp.stack(hs_list, 0)
    preds3, loss = _readout(hs, a2, out_W, out_b.reshape(1, 9))
    return preds3.reshape(STEPS, N), loss[0, 0]


# speedup vs baseline: 1.9628x; 1.3343x over previous
import jax
import jax.numpy as jnp
from jax import lax
from jax.experimental import pallas as pl
from jax.experimental.pallas import tpu as pltpu
from jax.experimental.pallas import tpu_sc as plsc

N = 10000
H = 84
E = 160000
STEPS = 4

NC = 2
NS = 16
NW = NC * NS
CHUNK = 128
CPW = 40
EPW = CHUNK * CPW
E_PAD = NW * EPW
NROWS = NW * CPW
RPS = 632
NM = RPS * NS

BN = 2000
BE = 2048
HP = 128


def _full(shape):
    return pl.BlockSpec(shape, lambda *_: tuple(0 for _ in shape))


def _input_body(q_ref, emb_ref, w0, b0, w1, b1, w2, b2, w3, b3,
                w0a, w0b, b0m, x_ref, p_ref, qq_ref):
    t = emb_ref[...]
    t = jnp.maximum(jnp.dot(t, w0[...], preferred_element_type=jnp.float32) + b0[...], 0.0)
    t = jnp.maximum(jnp.dot(t, w1[...], preferred_element_type=jnp.float32) + b1[...], 0.0)
    t = jnp.maximum(jnp.dot(t, w2[...], preferred_element_type=jnp.float32) + b2[...], 0.0)
    table = jnp.dot(t, w3[...], preferred_element_type=jnp.float32) + b3[...]
    io = lax.broadcasted_iota(jnp.int32, (BN, 9), 1)
    oh = (io == q_ref[...]).astype(jnp.float32)
    x = jnp.dot(oh, table, preferred_element_type=jnp.float32)
    x_ref[...] = x
    p_ref[...] = jnp.dot(x, w0a[...], preferred_element_type=jnp.float32) + b0m[...]
    qq_ref[...] = jnp.dot(x, w0b[...], preferred_element_type=jnp.float32)


def _input_mlp(q2, embed, iw, ib, w0a, w0b, b0m):
    f32 = jnp.float32
    return pl.pallas_call(
        _input_body,
        grid=(N // BN,),
        in_specs=[
            pl.BlockSpec((BN, 1), lambda i: (i, 0)),
            _full((9, 16)),
            _full((16, H)), _full((1, H)),
            _full((H, H)), _full((1, H)),
            _full((H, H)), _full((1, H)),
            _full((H, H)), _full((1, H)),
            _full((H, HP)), _full((H, HP)), _full((1, HP)),
        ],
        out_specs=[
            pl.BlockSpec((BN, H), lambda i: (i, 0)),
            pl.BlockSpec((BN, HP), lambda i: (i, 0)),
            pl.BlockSpec((BN, HP), lambda i: (i, 0)),
        ],
        out_shape=[jax.ShapeDtypeStruct((N, H), f32),
                   jax.ShapeDtypeStruct((N, HP), f32),
                   jax.ShapeDtypeStruct((N, HP), f32)],
    )(q2, embed, iw[0], ib[0], iw[1], ib[1], iw[2], ib[2], iw[3], ib[3],
      w0a, w0b, b0m)


def _edge_body(a_ref, b_ref, w1, b1, w2, b2, w3, b3, e_ref):
    t = jnp.maximum(a_ref[...] + b_ref[...], 0.0)
    t = jnp.maximum(jnp.dot(t, w1[...], preferred_element_type=jnp.float32) + b1[...], 0.0)
    t = jnp.maximum(jnp.dot(t, w2[...], preferred_element_type=jnp.float32) + b2[...], 0.0)
    e_ref[...] = jnp.dot(t, w3[...], preferred_element_type=jnp.float32) + b3[...]


def _edge_mlp(ea, eb, w1p, b1, w2, b2, w3p, b3p):
    return pl.pallas_call(
        _edge_body,
        grid=(E_PAD // BE,),
        in_specs=[
            pl.BlockSpec((BE, HP), lambda i: (i, 0)),
            pl.BlockSpec((BE, HP), lambda i: (i, 0)),
            _full((HP, H)), _full((1, H)),
            _full((H, H)), _full((1, H)),
            _full((H, HP)), _full((1, HP)),
        ],
        out_specs=pl.BlockSpec((BE, HP), lambda i: (i, 0)),
        out_shape=jax.ShapeDtypeStruct((E_PAD, HP), jnp.float32),
    )(ea, eb, w1p, b1, w2, b2, w3p, b3p)


def _sigmoid(x):
    return 1.0 / (1.0 + jnp.exp(-x))


def _lstm_body(x_ref, m0_ref, m1_ref, h_ref, c_ref,
               wxi, wmi, whi, wxf, wmf, whf, wxg, wmg, whg, wxo, wmo, who,
               w0a, w0b, b0m,
               h_out, c_out, p_out, q_out):
    x = x_ref[...]
    m = m0_ref[0] + m1_ref[0]
    h = h_ref[...]

    def gate(wx, wm, wh):
        return (jnp.dot(x, wx[...], preferred_element_type=jnp.float32)
                + jnp.dot(m, wm[...], preferred_element_type=jnp.float32)
                + jnp.dot(h, wh[...], preferred_element_type=jnp.float32))

    i_g = _sigmoid(gate(wxi, wmi, whi))
    f_g = _sigmoid(gate(wxf, wmf, whf))
    g_g = jnp.tanh(gate(wxg, wmg, whg))
    o_g = _sigmoid(gate(wxo, wmo, who))
    c_new = f_g * c_ref[...] + i_g * g_g
    h_new = o_g * jnp.tanh(c_new)
    h_out[...] = h_new
    c_out[...] = c_new
    p_out[...] = jnp.dot(h_new, w0a[...], preferred_element_type=jnp.float32) + b0m[...]
    q_out[...] = jnp.dot(h_new, w0b[...], preferred_element_type=jnp.float32)


def _lstm(x, mp, h, c, gw, w0a, w0b, b0m):
    f32 = jnp.float32
    return pl.pallas_call(
        _lstm_body,
        grid=(N // BN,),
        in_specs=[
            pl.BlockSpec((BN, H), lambda i: (i, 0)),
            pl.BlockSpec((1, BN, HP), lambda i: (0, i, 0)),
            pl.BlockSpec((1, BN, HP), lambda i: (1, i, 0)),
            pl.BlockSpec((BN, H), lambda i: (i, 0)),
            pl.BlockSpec((BN, H), lambda i: (i, 0)),
        ] + [_full((H, H)), _full((HP, H)), _full((H, H))] * 4
          + [_full((H, HP)), _full((H, HP)), _full((1, HP))],
        out_specs=[pl.BlockSpec((BN, H), lambda i: (i, 0))] * 2
          + [pl.BlockSpec((BN, HP), lambda i: (i, 0))] * 2,
        out_shape=[jax.ShapeDtypeStruct((N, H), f32)] * 2
          + [jax.ShapeDtypeStruct((N, HP), f32)] * 2,
    )(x, mp, mp, h, c, *gw, w0a, w0b, b0m)


def _readout_body(hs_ref, a_ref, w_ref, b_ref, preds_ref, loss_ref):
    s = pl.program_id(0)
    h = hs_ref[0]
    l = jnp.dot(h, w_ref[...], preferred_element_type=jnp.float32) + b_ref[...]
    mx = jnp.max(l, axis=1, keepdims=True)
    io = lax.broadcasted_iota(jnp.int32, (N, 9), 1)
    pred = jnp.min(jnp.where(l == mx, io, 9), axis=1)
    preds_ref[...] = pred.reshape(1, 1, N)
    lse = mx[:, 0] + jnp.log(jnp.sum(jnp.exp(l - mx), axis=1))
    l_lab = jnp.sum(jnp.where(io == a_ref[...], l, 0.0), axis=1)
    part = jnp.sum(lse - l_lab).reshape(1, 1)

    @pl.when(s == 0)
    def _():
        loss_ref[...] = jnp.zeros((1, 1), jnp.float32)

    loss_ref[...] += part

    @pl.when(s == STEPS - 1)
    def _():
        loss_ref[...] = loss_ref[...] / float(STEPS * N)


def _readout(hs, a2, out_w, out_b2):
    return pl.pallas_call(
        _readout_body,
        grid=(STEPS,),
        in_specs=[
            pl.BlockSpec((1, N, H), lambda s: (s, 0, 0)),
            pl.BlockSpec((N, 1), lambda s: (0, 0)),
            _full((H, 9)),
            _full((1, 9)),
        ],
        out_specs=[
            pl.BlockSpec((1, 1, N), lambda s: (s, 0, 0)),
            pl.BlockSpec((1, 1), lambda s: (0, 0)),
        ],
        out_shape=[
            jax.ShapeDtypeStruct((STEPS, 1, N), jnp.int32),
            jax.ShapeDtypeStruct((1, 1), jnp.float32),
        ],
    )(hs, a2, out_w, out_b2)


NBUF = 4
NG = 2 * CPW // NBUF


def _sc_gather_body(p_hbm, q_hbm, src_hbm, dst_hbm, ea_hbm, eb_hbm,
                    idx_s, idx_d, b0, b1, b2, b3,
                    g0, g1, g2, g3, w0, w1, w2, w3):
    wid = lax.axis_index("s") * NC + lax.axis_index("c")
    row0 = wid * CPW
    pltpu.sync_copy(src_hbm.at[pl.ds(row0, CPW)], idx_s)
    pltpu.sync_copy(dst_hbm.at[pl.ds(row0, CPW)], idx_d)

    bufs = [b0, b1, b2, b3]
    gsem = [g0, g1, g2, g3]
    wsem = [w0, w1, w2, w3]

    def fire_gather(b, g):
        j = g * (NBUF // 2) + b // 2
        if b % 2 == 0:
            pltpu.async_copy(p_hbm.at[idx_s.at[j]], bufs[b], gsem[b])
        else:
            pltpu.async_copy(q_hbm.at[idx_d.at[j]], bufs[b], gsem[b])

    def wait_gather(b):
        pltpu.make_async_copy(p_hbm.at[idx_s.at[0]], bufs[b], gsem[b]).wait()

    def fire_wb(b, g):
        j = g * (NBUF // 2) + b // 2
        off = (row0 + j) * CHUNK
        tgt = ea_hbm if b % 2 == 0 else eb_hbm
        pltpu.async_copy(bufs[b], tgt.at[pl.ds(off, CHUNK)], wsem[b])

    def wait_wb(b):
        pltpu.make_async_copy(bufs[b], ea_hbm.at[pl.ds(row0 * CHUNK, CHUNK)],
                              wsem[b]).wait()

    for b in range(NBUF):
        fire_gather(b, 0)

    def body(g, carry):
        for b in range(NBUF):
            wait_gather(b)
            fire_wb(b, g)

        @pl.when(g + 1 < NG)
        def _():
            for b in range(NBUF):
                wait_wb(b)
                fire_gather(b, g + 1)

        return carry

    lax.fori_loop(0, NG, body, 0)
    for b in range(NBUF):
        wait_wb(b)


def _sc_gather(p, q, src_r, dst_r):
    f32 = jnp.float32
    return pl.kernel(
        _sc_gather_body,
        [jax.ShapeDtypeStruct((E_PAD, HP), f32)] * 2,
        mesh=plsc.VectorSubcoreMesh(core_axis_name="c", subcore_axis_name="s"),
        scratch_types=[
            pltpu.VMEM((CPW, CHUNK), jnp.int32),
            pltpu.VMEM((CPW, CHUNK), jnp.int32),
        ] + [pltpu.VMEM((CHUNK, HP), f32)] * NBUF
          + [pltpu.SemaphoreType.DMA] * (2 * NBUF),
    )(p, q, src_r, dst_r)


def _sc_scatter_body(e_hbm, dst_hbm, z_hbm, out_hbm, msh, idx_d, buf, sem):
    cid = lax.axis_index("c")
    sid = lax.axis_index("s")
    wid = sid * NC + cid
    row0 = wid * CPW
    pltpu.sync_copy(z_hbm.at[pl.ds(sid * RPS, RPS)], msh.at[pl.ds(sid * RPS, RPS)])
    pltpu.sync_copy(dst_hbm.at[pl.ds(row0, CPW)], idx_d)
    plsc.subcore_barrier()

    def chunk(j, carry):
        off = (row0 + j) * CHUNK
        pltpu.sync_copy(e_hbm.at[pl.ds(off, CHUNK)], buf)
        pltpu.sync_copy(buf, msh.at[idx_d.at[j]], add=True)
        return carry

    lax.fori_loop(0, CPW, chunk, 0)
    plsc.subcore_barrier()
    pltpu.sync_copy(msh.at[pl.ds(sid * RPS, RPS)],
                    out_hbm.at[cid, pl.ds(sid * RPS, RPS)])


def _sc_scatter(e, dst_r, zrows):
    f32 = jnp.float32
    return pl.kernel(
        _sc_scatter_body,
        jax.ShapeDtypeStruct((NC, NM, HP), f32),
        mesh=plsc.VectorSubcoreMesh(core_axis_name="c", subcore_axis_name="s"),
        scratch_types=[
            pltpu.VMEM_SHARED((NM, HP), f32),
            pltpu.VMEM((CPW, CHUNK), jnp.int32),
            pltpu.VMEM((CHUNK, HP), f32),
            pltpu.SemaphoreType.DMA,
        ],
    )(e, dst_r, zrows)


def kernel(q, a, edge_index, embed, in_W0, in_b0, in_W1, in_b1, in_W2, in_b2,
           in_W3, in_b3, msg_W0, msg_b0, msg_W1, msg_b1, msg_W2, msg_b2,
           msg_W3, msg_b3, W_ih, W_hh, out_W, out_b):
    f32 = jnp.float32
    i32 = jnp.int32
    q2 = q.astype(i32).reshape(N, 1)
    a2 = a.astype(i32).reshape(N, 1)
    ei = edge_index.astype(i32)
    pad = E_PAD - E
    src_r = jnp.concatenate([ei[0], jnp.zeros((pad,), i32)]).reshape(NROWS, CHUNK)
    dstg_r = jnp.concatenate([ei[1], jnp.zeros((pad,), i32)]).reshape(NROWS, CHUNK)
    dsts_r = jnp.concatenate([ei[1], jnp.full((pad,), N, i32)]).reshape(NROWS, CHUNK)
    zrows = jnp.zeros((NM, HP), f32)

    def padc(w):
        return jnp.pad(w, ((0, 0), (0, HP - w.shape[1])))

    def padr(w):
        return jnp.pad(w, ((0, HP - w.shape[0]), (0, 0)))

    iw = [in_W0, in_W1, in_W2, in_W3]
    ib = [b.reshape(1, H) for b in (in_b0, in_b1, in_b2, in_b3)]
    w0a = padc(msg_W0[:H])
    w0b = padc(msg_W0[H:])
    b0m = padc(msg_b0.reshape(1, H))
    w1p = padr(msg_W1)
    b1m = msg_b1.reshape(1, H)
    b2m = msg_b2.reshape(1, H)
    w3p = padc(msg_W3)
    b3p = padc(msg_b3.reshape(1, H))

    gw = []
    for k in range(4):
        gw.append(W_ih[k * H:(k + 1) * H, :H].T)
        gw.append(padr(W_ih[k * H:(k + 1) * H, H:].T))
        gw.append(W_hh[k * H:(k + 1) * H, :].T)

    x, p, qq = _input_mlp(q2, embed, iw, ib, w0a, w0b, b0m)
    h = x
    c = jnp.zeros((N, H), f32)
    hs_list = []
    for _ in range(STEPS):
        ea, eb = _sc_gather(p, qq, src_r, dstg_r)
        e = _edge_mlp(ea, eb, w1p, b1m, msg_W2, b2m, w3p, b3p)
        mp = _sc_scatter(e, dsts_r, zrows)
        h, c, p, qq = _lstm(x, mp, h, c, gw, w0a, w0b, b0m)
        hs_list.append(h)
    hs = jnp.stack(hs_list, 0)
    preds3, loss = _readout(hs, a2, out_W, out_b.reshape(1, 9))
    return preds3.reshape(STEPS, N), loss[0, 0]


# speedup vs baseline: 2.0147x; 1.0265x over previous
import jax
import jax.numpy as jnp
from jax import lax
from jax.experimental import pallas as pl
from jax.experimental.pallas import tpu as pltpu
from jax.experimental.pallas import tpu_sc as plsc

N = 10000
H = 84
E = 160000
STEPS = 4

NC = 2
NS = 16
NW = NC * NS
CHUNK = 128
CPW = 40
EPW = CHUNK * CPW
E_PAD = NW * EPW
NROWS = NW * CPW
RPS = 632
NM = RPS * NS

BN = 2000
BE = 2048
HP = 128


def _full(shape):
    return pl.BlockSpec(shape, lambda *_: tuple(0 for _ in shape))


def _input_body(q_ref, emb_ref, w0, b0, w1, b1, w2, b2, w3, b3,
                w0a, w0b, b0m, x_ref, p_ref, qq_ref):
    t = emb_ref[...]
    t = jnp.maximum(jnp.dot(t, w0[...], preferred_element_type=jnp.float32) + b0[...], 0.0)
    t = jnp.maximum(jnp.dot(t, w1[...], preferred_element_type=jnp.float32) + b1[...], 0.0)
    t = jnp.maximum(jnp.dot(t, w2[...], preferred_element_type=jnp.float32) + b2[...], 0.0)
    table = jnp.dot(t, w3[...], preferred_element_type=jnp.float32) + b3[...]
    io = lax.broadcasted_iota(jnp.int32, (BN, 9), 1)
    oh = (io == q_ref[...]).astype(jnp.float32)
    x = jnp.dot(oh, table, preferred_element_type=jnp.float32)
    x_ref[...] = x
    p_ref[...] = jnp.dot(x, w0a[...], preferred_element_type=jnp.float32) + b0m[...]
    qq_ref[...] = jnp.dot(x, w0b[...], preferred_element_type=jnp.float32)


def _input_mlp(q2, embed, iw, ib, w0a, w0b, b0m):
    f32 = jnp.float32
    return pl.pallas_call(
        _input_body,
        grid=(N // BN,),
        in_specs=[
            pl.BlockSpec((BN, 1), lambda i: (i, 0)),
            _full((9, 16)),
            _full((16, H)), _full((1, H)),
            _full((H, H)), _full((1, H)),
            _full((H, H)), _full((1, H)),
            _full((H, H)), _full((1, H)),
            _full((H, HP)), _full((H, HP)), _full((1, HP)),
        ],
        out_specs=[
            pl.BlockSpec((BN, H), lambda i: (i, 0)),
            pl.BlockSpec((BN, HP), lambda i: (i, 0)),
            pl.BlockSpec((BN, HP), lambda i: (i, 0)),
        ],
        out_shape=[jax.ShapeDtypeStruct((N, H), f32),
                   jax.ShapeDtypeStruct((N, HP), f32),
                   jax.ShapeDtypeStruct((N, HP), f32)],
    )(q2, embed, iw[0], ib[0], iw[1], ib[1], iw[2], ib[2], iw[3], ib[3],
      w0a, w0b, b0m)


def _edge_body(a_ref, b_ref, w1, b1, w2, b2, w3, b3, e_ref):
    t = jnp.maximum(a_ref[...] + b_ref[...], 0.0)
    t = jnp.maximum(jnp.dot(t, w1[...], preferred_element_type=jnp.float32) + b1[...], 0.0)
    t = jnp.maximum(jnp.dot(t, w2[...], preferred_element_type=jnp.float32) + b2[...], 0.0)
    e_ref[...] = jnp.dot(t, w3[...], preferred_element_type=jnp.float32) + b3[...]


def _edge_mlp(ea, eb, w1p, b1, w2, b2, w3p, b3p):
    return pl.pallas_call(
        _edge_body,
        grid=(E_PAD // BE,),
        in_specs=[
            pl.BlockSpec((BE, HP), lambda i: (i, 0)),
            pl.BlockSpec((BE, HP), lambda i: (i, 0)),
            _full((HP, H)), _full((1, H)),
            _full((H, H)), _full((1, H)),
            _full((H, HP)), _full((1, HP)),
        ],
        out_specs=pl.BlockSpec((BE, HP), lambda i: (i, 0)),
        out_shape=jax.ShapeDtypeStruct((E_PAD, HP), jnp.float32),
    )(ea, eb, w1p, b1, w2, b2, w3p, b3p)


def _sigmoid(x):
    return 1.0 / (1.0 + jnp.exp(-x))


def _lstm_body(x_ref, m0_ref, m1_ref, h_ref, c_ref,
               wxi, wmi, whi, wxf, wmf, whf, wxg, wmg, whg, wxo, wmo, who,
               w0a, w0b, b0m,
               h_out, c_out, p_out, q_out):
    x = x_ref[...]
    m = m0_ref[0] + m1_ref[0]
    h = h_ref[...]

    def gate(wx, wm, wh):
        return (jnp.dot(x, wx[...], preferred_element_type=jnp.float32)
                + jnp.dot(m, wm[...], preferred_element_type=jnp.float32)
                + jnp.dot(h, wh[...], preferred_element_type=jnp.float32))

    i_g = _sigmoid(gate(wxi, wmi, whi))
    f_g = _sigmoid(gate(wxf, wmf, whf))
    g_g = jnp.tanh(gate(wxg, wmg, whg))
    o_g = _sigmoid(gate(wxo, wmo, who))
    c_new = f_g * c_ref[...] + i_g * g_g
    h_new = o_g * jnp.tanh(c_new)
    h_out[...] = h_new
    c_out[...] = c_new
    p_out[...] = jnp.dot(h_new, w0a[...], preferred_element_type=jnp.float32) + b0m[...]
    q_out[...] = jnp.dot(h_new, w0b[...], preferred_element_type=jnp.float32)


def _lstm(x, mp, h, c, gw, w0a, w0b, b0m):
    f32 = jnp.float32
    return pl.pallas_call(
        _lstm_body,
        grid=(N // BN,),
        in_specs=[
            pl.BlockSpec((BN, H), lambda i: (i, 0)),
            pl.BlockSpec((1, BN, HP), lambda i: (0, i, 0)),
            pl.BlockSpec((1, BN, HP), lambda i: (1, i, 0)),
            pl.BlockSpec((BN, H), lambda i: (i, 0)),
            pl.BlockSpec((BN, H), lambda i: (i, 0)),
        ] + [_full((H, H)), _full((HP, H)), _full((H, H))] * 4
          + [_full((H, HP)), _full((H, HP)), _full((1, HP))],
        out_specs=[pl.BlockSpec((BN, H), lambda i: (i, 0))] * 2
          + [pl.BlockSpec((BN, HP), lambda i: (i, 0))] * 2,
        out_shape=[jax.ShapeDtypeStruct((N, H), f32)] * 2
          + [jax.ShapeDtypeStruct((N, HP), f32)] * 2,
    )(x, mp, mp, h, c, *gw, w0a, w0b, b0m)


def _readout_body(hs_ref, a_ref, w_ref, b_ref, preds_ref, loss_ref):
    s = pl.program_id(0)
    h = hs_ref[0]
    l = jnp.dot(h, w_ref[...], preferred_element_type=jnp.float32) + b_ref[...]
    mx = jnp.max(l, axis=1, keepdims=True)
    io = lax.broadcasted_iota(jnp.int32, (N, 9), 1)
    pred = jnp.min(jnp.where(l == mx, io, 9), axis=1)
    preds_ref[...] = pred.reshape(1, 1, N)
    lse = mx[:, 0] + jnp.log(jnp.sum(jnp.exp(l - mx), axis=1))
    l_lab = jnp.sum(jnp.where(io == a_ref[...], l, 0.0), axis=1)
    part = jnp.sum(lse - l_lab).reshape(1, 1)

    @pl.when(s == 0)
    def _():
        loss_ref[...] = jnp.zeros((1, 1), jnp.float32)

    loss_ref[...] += part

    @pl.when(s == STEPS - 1)
    def _():
        loss_ref[...] = loss_ref[...] / float(STEPS * N)


def _readout(hs, a2, out_w, out_b2):
    return pl.pallas_call(
        _readout_body,
        grid=(STEPS,),
        in_specs=[
            pl.BlockSpec((1, N, H), lambda s: (s, 0, 0)),
            pl.BlockSpec((N, 1), lambda s: (0, 0)),
            _full((H, 9)),
            _full((1, 9)),
        ],
        out_specs=[
            pl.BlockSpec((1, 1, N), lambda s: (s, 0, 0)),
            pl.BlockSpec((1, 1), lambda s: (0, 0)),
        ],
        out_shape=[
            jax.ShapeDtypeStruct((STEPS, 1, N), jnp.int32),
            jax.ShapeDtypeStruct((1, 1), jnp.float32),
        ],
    )(hs, a2, out_w, out_b2)


NBUF = 4
NG = 2 * CPW // NBUF


def _sc_gather_body(p_hbm, q_hbm, src_hbm, dst_hbm, ea_hbm, eb_hbm,
                    idx_s, idx_d, b0, b1, b2, b3,
                    g0, g1, g2, g3, w0, w1, w2, w3):
    wid = lax.axis_index("s") * NC + lax.axis_index("c")
    row0 = wid * CPW
    pltpu.sync_copy(src_hbm.at[pl.ds(row0, CPW)], idx_s)
    pltpu.sync_copy(dst_hbm.at[pl.ds(row0, CPW)], idx_d)

    bufs = [b0, b1, b2, b3]
    gsem = [g0, g1, g2, g3]
    wsem = [w0, w1, w2, w3]

    def fire_gather(b, g):
        j = g * (NBUF // 2) + b // 2
        if b % 2 == 0:
            pltpu.async_copy(p_hbm.at[idx_s.at[j]], bufs[b], gsem[b])
        else:
            pltpu.async_copy(q_hbm.at[idx_d.at[j]], bufs[b], gsem[b])

    def wait_gather(b):
        pltpu.make_async_copy(p_hbm.at[idx_s.at[0]], bufs[b], gsem[b]).wait()

    def fire_wb(b, g):
        j = g * (NBUF // 2) + b // 2
        off = (row0 + j) * CHUNK
        tgt = ea_hbm if b % 2 == 0 else eb_hbm
        pltpu.async_copy(bufs[b], tgt.at[pl.ds(off, CHUNK)], wsem[b])

    def wait_wb(b):
        pltpu.make_async_copy(bufs[b], ea_hbm.at[pl.ds(row0 * CHUNK, CHUNK)],
                              wsem[b]).wait()

    for b in range(NBUF):
        fire_gather(b, 0)

    def body(g, carry):
        for b in range(NBUF):
            wait_gather(b)
            fire_wb(b, g)

        @pl.when(g + 1 < NG)
        def _():
            for b in range(NBUF):
                wait_wb(b)
                fire_gather(b, g + 1)

        return carry

    lax.fori_loop(0, NG, body, 0)
    for b in range(NBUF):
        wait_wb(b)


def _sc_gather(p, q, src_r, dst_r):
    f32 = jnp.float32
    return pl.kernel(
        _sc_gather_body,
        [jax.ShapeDtypeStruct((E_PAD, HP), f32)] * 2,
        mesh=plsc.VectorSubcoreMesh(core_axis_name="c", subcore_axis_name="s"),
        scratch_types=[
            pltpu.VMEM((CPW, CHUNK), jnp.int32),
            pltpu.VMEM((CPW, CHUNK), jnp.int32),
        ] + [pltpu.VMEM((CHUNK, HP), f32)] * NBUF
          + [pltpu.SemaphoreType.DMA] * (2 * NBUF),
    )(p, q, src_r, dst_r)


SBUF = 2
SG = CPW // SBUF


def _sc_scatter_body(e_hbm, dst_hbm, z_hbm, out_hbm, msh, idx_d,
                     b0, b1, r0, r1, a0, a1):
    cid = lax.axis_index("c")
    sid = lax.axis_index("s")
    wid = sid * NC + cid
    row0 = wid * CPW
    pltpu.sync_copy(z_hbm.at[pl.ds(sid * RPS, RPS)], msh.at[pl.ds(sid * RPS, RPS)])
    pltpu.sync_copy(dst_hbm.at[pl.ds(row0, CPW)], idx_d)
    plsc.subcore_barrier()

    bufs = [b0, b1]
    rsem = [r0, r1]
    asem = [a0, a1]

    def fire_read(b, g):
        j = g * SBUF + b
        off = (row0 + j) * CHUNK
        pltpu.async_copy(e_hbm.at[pl.ds(off, CHUNK)], bufs[b], rsem[b])

    def wait_read(b):
        pltpu.make_async_copy(e_hbm.at[pl.ds(row0 * CHUNK, CHUNK)], bufs[b],
                              rsem[b]).wait()

    def fire_add(b, g):
        j = g * SBUF + b
        pltpu.async_copy(bufs[b], msh.at[idx_d.at[j]], asem[b], add=True)

    def wait_add(b):
        pltpu.make_async_copy(bufs[b], msh.at[idx_d.at[0]], asem[b]).wait()

    for b in range(SBUF):
        fire_read(b, 0)

    def body(g, carry):
        for b in range(SBUF):
            wait_read(b)
            fire_add(b, g)

        @pl.when(g + 1 < SG)
        def _():
            for b in range(SBUF):
                wait_add(b)
                fire_read(b, g + 1)

        return carry

    lax.fori_loop(0, SG, body, 0)
    for b in range(SBUF):
        wait_add(b)
    plsc.subcore_barrier()
    pltpu.sync_copy(msh.at[pl.ds(sid * RPS, RPS)],
                    out_hbm.at[cid, pl.ds(sid * RPS, RPS)])


def _sc_scatter(e, dst_r, zrows):
    f32 = jnp.float32
    return pl.kernel(
        _sc_scatter_body,
        jax.ShapeDtypeStruct((NC, NM, HP), f32),
        mesh=plsc.VectorSubcoreMesh(core_axis_name="c", subcore_axis_name="s"),
        scratch_types=[
            pltpu.VMEM_SHARED((NM, HP), f32),
            pltpu.VMEM((CPW, CHUNK), jnp.int32),
        ] + [pltpu.VMEM((CHUNK, HP), f32)] * SBUF
          + [pltpu.SemaphoreType.DMA] * (2 * SBUF),
    )(e, dst_r, zrows)


def kernel(q, a, edge_index, embed, in_W0, in_b0, in_W1, in_b1, in_W2, in_b2,
           in_W3, in_b3, msg_W0, msg_b0, msg_W1, msg_b1, msg_W2, msg_b2,
           msg_W3, msg_b3, W_ih, W_hh, out_W, out_b):
    f32 = jnp.float32
    i32 = jnp.int32
    q2 = q.astype(i32).reshape(N, 1)
    a2 = a.astype(i32).reshape(N, 1)
    ei = edge_index.astype(i32)
    pad = E_PAD - E
    src_r = jnp.concatenate([ei[0], jnp.zeros((pad,), i32)]).reshape(NROWS, CHUNK)
    dstg_r = jnp.concatenate([ei[1], jnp.zeros((pad,), i32)]).reshape(NROWS, CHUNK)
    dsts_r = jnp.concatenate([ei[1], jnp.full((pad,), N, i32)]).reshape(NROWS, CHUNK)
    zrows = jnp.zeros((NM, HP), f32)

    def padc(w):
        return jnp.pad(w, ((0, 0), (0, HP - w.shape[1])))

    def padr(w):
        return jnp.pad(w, ((0, HP - w.shape[0]), (0, 0)))

    iw = [in_W0, in_W1, in_W2, in_W3]
    ib = [b.reshape(1, H) for b in (in_b0, in_b1, in_b2, in_b3)]
    w0a = padc(msg_W0[:H])
    w0b = padc(msg_W0[H:])
    b0m = padc(msg_b0.reshape(1, H))
    w1p = padr(msg_W1)
    b1m = msg_b1.reshape(1, H)
    b2m = msg_b2.reshape(1, H)
    w3p = padc(msg_W3)
    b3p = padc(msg_b3.reshape(1, H))

    gw = []
    for k in range(4):
        gw.append(W_ih[k * H:(k + 1) * H, :H].T)
        gw.append(padr(W_ih[k * H:(k + 1) * H, H:].T))
        gw.append(W_hh[k * H:(k + 1) * H, :].T)

    x, p, qq = _input_mlp(q2, embed, iw, ib, w0a, w0b, b0m)
    h = x
    c = jnp.zeros((N, H), f32)
    hs_list = []
    for _ in range(STEPS):
        ea, eb = _sc_gather(p, qq, src_r, dstg_r)
        e = _edge_mlp(ea, eb, w1p, b1m, msg_W2, b2m, w3p, b3p)
        mp = _sc_scatter(e, dsts_r, zrows)
        h, c, p, qq = _lstm(x, mp, h, c, gw, w0a, w0b, b0m)
        hs_list.append(h)
    hs = jnp.stack(hs_list, 0)
    preds3, loss = _readout(hs, a2, out_W, out_b.reshape(1, 9))
    return preds3.reshape(STEPS, N), loss[0, 0]


# speedup vs baseline: 3.4702x; 1.7224x over previous
import jax
import jax.numpy as jnp
from jax import lax
from jax.experimental import pallas as pl
from jax.experimental.pallas import tpu as pltpu
from jax.experimental.pallas import tpu_sc as plsc

N = 10000
H = 84
E = 160000
STEPS = 4

NC = 2
NS = 16
NW = NC * NS
CHUNK = 128
CPW = 40
EPW = CHUNK * CPW
E_PAD = NW * EPW
NROWS = NW * CPW
RPS = 632
NM = RPS * NS

BN = 2000
BE = 2048
HP = 128


def _full(shape):
    return pl.BlockSpec(shape, lambda *_: tuple(0 for _ in shape))


def _input_body(q_ref, emb_ref, w0, b0, w1, b1, w2, b2, w3, b3,
                w0a, w0b, b0m, x_ref, p_ref, qq_ref):
    t = emb_ref[...]
    t = jnp.maximum(jnp.dot(t, w0[...], preferred_element_type=jnp.float32) + b0[...], 0.0)
    t = jnp.maximum(jnp.dot(t, w1[...], preferred_element_type=jnp.float32) + b1[...], 0.0)
    t = jnp.maximum(jnp.dot(t, w2[...], preferred_element_type=jnp.float32) + b2[...], 0.0)
    table = jnp.dot(t, w3[...], preferred_element_type=jnp.float32) + b3[...]
    io = lax.broadcasted_iota(jnp.int32, (BN, 9), 1)
    oh = (io == q_ref[...]).astype(jnp.float32)
    x = jnp.dot(oh, table, preferred_element_type=jnp.float32)
    x_ref[...] = x
    p_ref[...] = jnp.dot(x, w0a[...], preferred_element_type=jnp.float32) + b0m[...]
    qq_ref[...] = jnp.dot(x, w0b[...], preferred_element_type=jnp.float32)


def _input_mlp(q2, embed, iw, ib, w0a, w0b, b0m):
    f32 = jnp.float32
    return pl.pallas_call(
        _input_body,
        grid=(N // BN,),
        in_specs=[
            pl.BlockSpec((BN, 1), lambda i: (i, 0)),
            _full((9, 16)),
            _full((16, H)), _full((1, H)),
            _full((H, H)), _full((1, H)),
            _full((H, H)), _full((1, H)),
            _full((H, H)), _full((1, H)),
            _full((H, HP)), _full((H, HP)), _full((1, HP)),
        ],
        out_specs=[
            pl.BlockSpec((BN, H), lambda i: (i, 0)),
            pl.BlockSpec((BN, HP), lambda i: (i, 0)),
            pl.BlockSpec((BN, HP), lambda i: (i, 0)),
        ],
        out_shape=[jax.ShapeDtypeStruct((N, H), f32),
                   jax.ShapeDtypeStruct((N, HP), f32),
                   jax.ShapeDtypeStruct((N, HP), f32)],
    )(q2, embed, iw[0], ib[0], iw[1], ib[1], iw[2], ib[2], iw[3], ib[3],
      w0a, w0b, b0m)


def _edge_body(a_ref, b_ref, w1, b1, w2, b2, w3, b3, e_ref):
    t = jnp.maximum(a_ref[...] + b_ref[...], 0.0)
    t = jnp.maximum(jnp.dot(t, w1[...], preferred_element_type=jnp.float32) + b1[...], 0.0)
    t = jnp.maximum(jnp.dot(t, w2[...], preferred_element_type=jnp.float32) + b2[...], 0.0)
    e_ref[...] = jnp.dot(t, w3[...], preferred_element_type=jnp.float32) + b3[...]


def _edge_mlp(ea, eb, w1p, b1, w2, b2, w3p, b3p):
    return pl.pallas_call(
        _edge_body,
        grid=(E_PAD // BE,),
        in_specs=[
            pl.BlockSpec((BE, HP), lambda i: (i, 0)),
            pl.BlockSpec((BE, HP), lambda i: (i, 0)),
            _full((HP, H)), _full((1, H)),
            _full((H, H)), _full((1, H)),
            _full((H, HP)), _full((1, HP)),
        ],
        out_specs=pl.BlockSpec((BE, HP), lambda i: (i, 0)),
        out_shape=jax.ShapeDtypeStruct((E_PAD, HP), jnp.float32),
    )(ea, eb, w1p, b1, w2, b2, w3p, b3p)


def _sigmoid(x):
    return 1.0 / (1.0 + jnp.exp(-x))


def _lstm_body(x_ref, m0_ref, m1_ref, h_ref, c_ref,
               wxi, wmi, whi, wxf, wmf, whf, wxg, wmg, whg, wxo, wmo, who,
               w0a, w0b, b0m,
               h_out, c_out, p_out, q_out):
    x = x_ref[...]
    m = m0_ref[0] + m1_ref[0]
    h = h_ref[...]

    def gate(wx, wm, wh):
        return (jnp.dot(x, wx[...], preferred_element_type=jnp.float32)
                + jnp.dot(m, wm[...], preferred_element_type=jnp.float32)
                + jnp.dot(h, wh[...], preferred_element_type=jnp.float32))

    i_g = _sigmoid(gate(wxi, wmi, whi))
    f_g = _sigmoid(gate(wxf, wmf, whf))
    g_g = jnp.tanh(gate(wxg, wmg, whg))
    o_g = _sigmoid(gate(wxo, wmo, who))
    c_new = f_g * c_ref[...] + i_g * g_g
    h_new = o_g * jnp.tanh(c_new)
    h_out[...] = h_new
    c_out[...] = c_new
    p_out[...] = jnp.dot(h_new, w0a[...], preferred_element_type=jnp.float32) + b0m[...]
    q_out[...] = jnp.dot(h_new, w0b[...], preferred_element_type=jnp.float32)


def _lstm(x, mp, h, c, gw, w0a, w0b, b0m):
    f32 = jnp.float32
    return pl.pallas_call(
        _lstm_body,
        grid=(N // BN,),
        in_specs=[
            pl.BlockSpec((BN, H), lambda i: (i, 0)),
            pl.BlockSpec((1, BN, HP), lambda i: (0, i, 0)),
            pl.BlockSpec((1, BN, HP), lambda i: (1, i, 0)),
            pl.BlockSpec((BN, H), lambda i: (i, 0)),
            pl.BlockSpec((BN, H), lambda i: (i, 0)),
        ] + [_full((H, H)), _full((HP, H)), _full((H, H))] * 4
          + [_full((H, HP)), _full((H, HP)), _full((1, HP))],
        out_specs=[pl.BlockSpec((BN, H), lambda i: (i, 0))] * 2
          + [pl.BlockSpec((BN, HP), lambda i: (i, 0))] * 2,
        out_shape=[jax.ShapeDtypeStruct((N, H), f32)] * 2
          + [jax.ShapeDtypeStruct((N, HP), f32)] * 2,
    )(x, mp, mp, h, c, *gw, w0a, w0b, b0m)


def _readout_body(hs_ref, a_ref, w_ref, b_ref, preds_ref, loss_ref):
    s = pl.program_id(0)
    h = hs_ref[0]
    l = jnp.dot(h, w_ref[...], preferred_element_type=jnp.float32) + b_ref[...]
    mx = jnp.max(l, axis=1, keepdims=True)
    io = lax.broadcasted_iota(jnp.int32, (N, 9), 1)
    pred = jnp.min(jnp.where(l == mx, io, 9), axis=1)
    preds_ref[...] = pred.reshape(1, 1, N)
    lse = mx[:, 0] + jnp.log(jnp.sum(jnp.exp(l - mx), axis=1))
    l_lab = jnp.sum(jnp.where(io == a_ref[...], l, 0.0), axis=1)
    part = jnp.sum(lse - l_lab).reshape(1, 1)

    @pl.when(s == 0)
    def _():
        loss_ref[...] = jnp.zeros((1, 1), jnp.float32)

    loss_ref[...] += part

    @pl.when(s == STEPS - 1)
    def _():
        loss_ref[...] = loss_ref[...] / float(STEPS * N)


def _readout(hs, a2, out_w, out_b2):
    return pl.pallas_call(
        _readout_body,
        grid=(STEPS,),
        in_specs=[
            pl.BlockSpec((1, N, H), lambda s: (s, 0, 0)),
            pl.BlockSpec((N, 1), lambda s: (0, 0)),
            _full((H, 9)),
            _full((1, 9)),
        ],
        out_specs=[
            pl.BlockSpec((1, 1, N), lambda s: (s, 0, 0)),
            pl.BlockSpec((1, 1), lambda s: (0, 0)),
        ],
        out_shape=[
            jax.ShapeDtypeStruct((STEPS, 1, N), jnp.int32),
            jax.ShapeDtypeStruct((1, 1), jnp.float32),
        ],
    )(hs, a2, out_w, out_b2)


NBUF = 2
GCPW = E_PAD // (NS * CHUNK)
NG = GCPW // NBUF

TRS = 624


def _sc_gather_body(p_hbm, q_hbm, src_hbm, dst_hbm, ea_hbm, eb_hbm,
                    table, idx_v, b0, b1, g0, g1, w0, w1):
    cid = lax.axis_index("c")
    sid = lax.axis_index("s")
    bufs = [b0, b1]
    gsem = [g0, g1]
    wsem = [w0, w1]

    def run(tab_hbm, idx_hbm, out_hbm):
        r0 = sid * TRS
        @pl.when(sid < NS - 1)
        def _():
            pltpu.sync_copy(tab_hbm.at[pl.ds(r0, TRS)], table.at[pl.ds(r0, TRS)])
        @pl.when(sid == NS - 1)
        def _():
            rem = N - (NS - 1) * TRS
            pltpu.sync_copy(tab_hbm.at[pl.ds((NS - 1) * TRS, rem)],
                            table.at[pl.ds((NS - 1) * TRS, rem)])
        pltpu.sync_copy(idx_hbm.at[pl.ds(sid * GCPW, GCPW)], idx_v)
        plsc.subcore_barrier()

        def fire_gather(b, g):
            j = g * NBUF + b
            pltpu.async_copy(table.at[idx_v.at[j]], bufs[b], gsem[b])

        def wait_gather(b):
            pltpu.make_async_copy(table.at[idx_v.at[0]], bufs[b], gsem[b]).wait()

        def fire_wb(b, g):
            j = g * NBUF + b
            off = (sid * GCPW + j) * CHUNK
            pltpu.async_copy(bufs[b], out_hbm.at[pl.ds(off, CHUNK)], wsem[b])

        def wait_wb(b):
            pltpu.make_async_copy(bufs[b], out_hbm.at[pl.ds(0, CHUNK)],
                                  wsem[b]).wait()

        for b in range(NBUF):
            fire_gather(b, 0)

        def body(g, carry):
            for b in range(NBUF):
                wait_gather(b)
                fire_wb(b, g)

            @pl.when(g + 1 < NG)
            def _():
                for b in range(NBUF):
                    wait_wb(b)
                    fire_gather(b, g + 1)

            return carry

        lax.fori_loop(0, NG, body, 0)
        for b in range(NBUF):
            wait_wb(b)

    @pl.when(cid == 0)
    def _():
        run(p_hbm, src_hbm, ea_hbm)

    @pl.when(cid == 1)
    def _():
        run(q_hbm, dst_hbm, eb_hbm)


def _sc_gather(p, q, src_r, dst_r):
    f32 = jnp.float32
    return pl.kernel(
        _sc_gather_body,
        [jax.ShapeDtypeStruct((E_PAD, HP), f32)] * 2,
        mesh=plsc.VectorSubcoreMesh(core_axis_name="c", subcore_axis_name="s"),
        scratch_types=[
            pltpu.VMEM_SHARED((N, HP), f32),
            pltpu.VMEM((GCPW, CHUNK), jnp.int32),
        ] + [pltpu.VMEM((CHUNK, HP), f32)] * NBUF
          + [pltpu.SemaphoreType.DMA] * (2 * NBUF),
    )(p, q, src_r, dst_r)


SBUF = 2
SG = CPW // SBUF


def _sc_scatter_body(e_hbm, dst_hbm, z_hbm, out_hbm, msh, idx_d,
                     b0, b1, r0, r1, a0, a1):
    cid = lax.axis_index("c")
    sid = lax.axis_index("s")
    wid = sid * NC + cid
    row0 = wid * CPW
    pltpu.sync_copy(z_hbm.at[pl.ds(sid * RPS, RPS)], msh.at[pl.ds(sid * RPS, RPS)])
    pltpu.sync_copy(dst_hbm.at[pl.ds(row0, CPW)], idx_d)
    plsc.subcore_barrier()

    bufs = [b0, b1]
    rsem = [r0, r1]
    asem = [a0, a1]

    def fire_read(b, g):
        j = g * SBUF + b
        off = (row0 + j) * CHUNK
        pltpu.async_copy(e_hbm.at[pl.ds(off, CHUNK)], bufs[b], rsem[b])

    def wait_read(b):
        pltpu.make_async_copy(e_hbm.at[pl.ds(row0 * CHUNK, CHUNK)], bufs[b],
                              rsem[b]).wait()

    def fire_add(b, g):
        j = g * SBUF + b
        pltpu.async_copy(bufs[b], msh.at[idx_d.at[j]], asem[b], add=True)

    def wait_add(b):
        pltpu.make_async_copy(bufs[b], msh.at[idx_d.at[0]], asem[b]).wait()

    for b in range(SBUF):
        fire_read(b, 0)

    def body(g, carry):
        for b in range(SBUF):
            wait_read(b)
            fire_add(b, g)

        @pl.when(g + 1 < SG)
        def _():
            for b in range(SBUF):
                wait_add(b)
                fire_read(b, g + 1)

        return carry

    lax.fori_loop(0, SG, body, 0)
    for b in range(SBUF):
        wait_add(b)
    plsc.subcore_barrier()
    pltpu.sync_copy(msh.at[pl.ds(sid * RPS, RPS)],
                    out_hbm.at[cid, pl.ds(sid * RPS, RPS)])


def _sc_scatter(e, dst_r, zrows):
    f32 = jnp.float32
    return pl.kernel(
        _sc_scatter_body,
        jax.ShapeDtypeStruct((NC, NM, HP), f32),
        mesh=plsc.VectorSubcoreMesh(core_axis_name="c", subcore_axis_name="s"),
        scratch_types=[
            pltpu.VMEM_SHARED((NM, HP), f32),
            pltpu.VMEM((CPW, CHUNK), jnp.int32),
        ] + [pltpu.VMEM((CHUNK, HP), f32)] * SBUF
          + [pltpu.SemaphoreType.DMA] * (2 * SBUF),
    )(e, dst_r, zrows)


def kernel(q, a, edge_index, embed, in_W0, in_b0, in_W1, in_b1, in_W2, in_b2,
           in_W3, in_b3, msg_W0, msg_b0, msg_W1, msg_b1, msg_W2, msg_b2,
           msg_W3, msg_b3, W_ih, W_hh, out_W, out_b):
    f32 = jnp.float32
    i32 = jnp.int32
    q2 = q.astype(i32).reshape(N, 1)
    a2 = a.astype(i32).reshape(N, 1)
    ei = edge_index.astype(i32)
    pad = E_PAD - E
    src_r = jnp.concatenate([ei[0], jnp.zeros((pad,), i32)]).reshape(NROWS, CHUNK)
    dstg_r = jnp.concatenate([ei[1], jnp.zeros((pad,), i32)]).reshape(NROWS, CHUNK)
    dsts_r = jnp.concatenate([ei[1], jnp.full((pad,), N, i32)]).reshape(NROWS, CHUNK)
    zrows = jnp.zeros((NM, HP), f32)

    def padc(w):
        return jnp.pad(w, ((0, 0), (0, HP - w.shape[1])))

    def padr(w):
        return jnp.pad(w, ((0, HP - w.shape[0]), (0, 0)))

    iw = [in_W0, in_W1, in_W2, in_W3]
    ib = [b.reshape(1, H) for b in (in_b0, in_b1, in_b2, in_b3)]
    w0a = padc(msg_W0[:H])
    w0b = padc(msg_W0[H:])
    b0m = padc(msg_b0.reshape(1, H))
    w1p = padr(msg_W1)
    b1m = msg_b1.reshape(1, H)
    b2m = msg_b2.reshape(1, H)
    w3p = padc(msg_W3)
    b3p = padc(msg_b3.reshape(1, H))

    gw = []
    for k in range(4):
        gw.append(W_ih[k * H:(k + 1) * H, :H].T)
        gw.append(padr(W_ih[k * H:(k + 1) * H, H:].T))
        gw.append(W_hh[k * H:(k + 1) * H, :].T)

    x, p, qq = _input_mlp(q2, embed, iw, ib, w0a, w0b, b0m)
    h = x
    c = jnp.zeros((N, H), f32)
    hs_list = []
    for _ in range(STEPS):
        ea, eb = _sc_gather(p, qq, src_r, dstg_r)
        e = _edge_mlp(ea, eb, w1p, b1m, msg_W2, b2m, w3p, b3p)
        mp = _sc_scatter(e, dsts_r, zrows)
        h, c, p, qq = _lstm(x, mp, h, c, gw, w0a, w0b, b0m)
        hs_list.append(h)
    hs = jnp.stack(hs_list, 0)
    preds3, loss = _readout(hs, a2, out_W, out_b.reshape(1, 9))
    return preds3.reshape(STEPS, N), loss[0, 0]


# speedup vs baseline: 4.0188x; 1.1581x over previous
import functools

import jax
import jax.numpy as jnp
from jax import lax
from jax.experimental import pallas as pl
from jax.experimental.pallas import tpu as pltpu
from jax.experimental.pallas import tpu_sc as plsc

N = 10000
H = 84
E = 160000
STEPS = 4

NC = 2
NS = 16
NW = NC * NS
CHUNK = 128
CPW = 40
EPW = CHUNK * CPW
E_PAD = NW * EPW
NROWS = NW * CPW
RPS = 632
NM = RPS * NS

BN = 2000
BE = 2048
HP = 128


def _full(shape):
    return pl.BlockSpec(shape, lambda *_: tuple(0 for _ in shape))


def _input_body(q_ref, emb_ref, w0, b0, w1, b1, w2, b2, w3, b3,
                w0a, w0b, b0m, x_ref, p_ref, qq_ref):
    t = emb_ref[...]
    t = jnp.maximum(jnp.dot(t, w0[...], preferred_element_type=jnp.float32) + b0[...], 0.0)
    t = jnp.maximum(jnp.dot(t, w1[...], preferred_element_type=jnp.float32) + b1[...], 0.0)
    t = jnp.maximum(jnp.dot(t, w2[...], preferred_element_type=jnp.float32) + b2[...], 0.0)
    table = jnp.dot(t, w3[...], preferred_element_type=jnp.float32) + b3[...]
    io = lax.broadcasted_iota(jnp.int32, (BN, 9), 1)
    oh = (io == q_ref[...]).astype(jnp.float32)
    x = jnp.dot(oh, table, preferred_element_type=jnp.float32)
    x_ref[...] = x
    p_ref[...] = jnp.dot(x, w0a[...], preferred_element_type=jnp.float32) + b0m[...]
    qq_ref[...] = jnp.dot(x, w0b[...], preferred_element_type=jnp.float32)


def _input_mlp(q2, embed, iw, ib, w0a, w0b, b0m):
    f32 = jnp.float32
    return pl.pallas_call(
        _input_body,
        grid=(N // BN,),
        in_specs=[
            pl.BlockSpec((BN, 1), lambda i: (i, 0)),
            _full((9, 16)),
            _full((16, H)), _full((1, H)),
            _full((H, H)), _full((1, H)),
            _full((H, H)), _full((1, H)),
            _full((H, H)), _full((1, H)),
            _full((H, HP)), _full((H, HP)), _full((1, HP)),
        ],
        out_specs=[
            pl.BlockSpec((BN, H), lambda i: (i, 0)),
            pl.BlockSpec((BN, HP), lambda i: (i, 0)),
            pl.BlockSpec((BN, HP), lambda i: (i, 0)),
        ],
        out_shape=[jax.ShapeDtypeStruct((N, H), f32),
                   jax.ShapeDtypeStruct((N, HP), f32),
                   jax.ShapeDtypeStruct((N, HP), f32)],
    )(q2, embed, iw[0], ib[0], iw[1], ib[1], iw[2], ib[2], iw[3], ib[3],
      w0a, w0b, b0m)


def _edge_body(a_ref, b_ref, w1, b1, w2, b2, w3, b3, e_ref):
    t = jnp.maximum(a_ref[...] + b_ref[...], 0.0)
    t = jnp.maximum(jnp.dot(t, w1[...], preferred_element_type=jnp.float32) + b1[...], 0.0)
    t = jnp.maximum(jnp.dot(t, w2[...], preferred_element_type=jnp.float32) + b2[...], 0.0)
    e_ref[...] = jnp.dot(t, w3[...], preferred_element_type=jnp.float32) + b3[...]


def _edge_mlp(ea, eb, w1p, b1, w2, b2, w3p, b3p):
    return pl.pallas_call(
        _edge_body,
        grid=(EH // BE,),
        in_specs=[
            pl.BlockSpec((BE, HP), lambda i: (i, 0)),
            pl.BlockSpec((BE, HP), lambda i: (i, 0)),
            _full((HP, H)), _full((1, H)),
            _full((H, H)), _full((1, H)),
            _full((H, HP)), _full((1, HP)),
        ],
        out_specs=pl.BlockSpec((BE, HP), lambda i: (i, 0)),
        out_shape=jax.ShapeDtypeStruct((EH, HP), jnp.float32),
    )(ea, eb, w1p, b1, w2, b2, w3p, b3p)


def _sigmoid(x):
    return 1.0 / (1.0 + jnp.exp(-x))


def _lstm_body(x_ref, m0_ref, m1_ref, m2_ref, m3_ref, h_ref, c_ref,
               wxi, wmi, whi, wxf, wmf, whf, wxg, wmg, whg, wxo, wmo, who,
               w0a, w0b, b0m,
               h_out, c_out, p_out, q_out):
    x = x_ref[...]
    m = (m0_ref[0] + m1_ref[0]) + (m2_ref[0] + m3_ref[0])
    h = h_ref[...]

    def gate(wx, wm, wh):
        return (jnp.dot(x, wx[...], preferred_element_type=jnp.float32)
                + jnp.dot(m, wm[...], preferred_element_type=jnp.float32)
                + jnp.dot(h, wh[...], preferred_element_type=jnp.float32))

    i_g = _sigmoid(gate(wxi, wmi, whi))
    f_g = _sigmoid(gate(wxf, wmf, whf))
    g_g = jnp.tanh(gate(wxg, wmg, whg))
    o_g = _sigmoid(gate(wxo, wmo, who))
    c_new = f_g * c_ref[...] + i_g * g_g
    h_new = o_g * jnp.tanh(c_new)
    h_out[...] = h_new
    c_out[...] = c_new
    p_out[...] = jnp.dot(h_new, w0a[...], preferred_element_type=jnp.float32) + b0m[...]
    q_out[...] = jnp.dot(h_new, w0b[...], preferred_element_type=jnp.float32)


def _lstm(x, mpa, mpb, h, c, gw, w0a, w0b, b0m):
    f32 = jnp.float32
    return pl.pallas_call(
        _lstm_body,
        grid=(N // BN,),
        in_specs=[
            pl.BlockSpec((BN, H), lambda i: (i, 0)),
            pl.BlockSpec((1, BN, HP), lambda i: (0, i, 0)),
            pl.BlockSpec((1, BN, HP), lambda i: (1, i, 0)),
            pl.BlockSpec((1, BN, HP), lambda i: (0, i, 0)),
            pl.BlockSpec((1, BN, HP), lambda i: (1, i, 0)),
            pl.BlockSpec((BN, H), lambda i: (i, 0)),
            pl.BlockSpec((BN, H), lambda i: (i, 0)),
        ] + [_full((H, H)), _full((HP, H)), _full((H, H))] * 4
          + [_full((H, HP)), _full((H, HP)), _full((1, HP))],
        out_specs=[pl.BlockSpec((BN, H), lambda i: (i, 0))] * 2
          + [pl.BlockSpec((BN, HP), lambda i: (i, 0))] * 2,
        out_shape=[jax.ShapeDtypeStruct((N, H), f32)] * 2
          + [jax.ShapeDtypeStruct((N, HP), f32)] * 2,
    )(x, mpa, mpa, mpb, mpb, h, c, *gw, w0a, w0b, b0m)


def _readout_body(hs_ref, a_ref, w_ref, b_ref, preds_ref, loss_ref):
    s = pl.program_id(0)
    h = hs_ref[0]
    l = jnp.dot(h, w_ref[...], preferred_element_type=jnp.float32) + b_ref[...]
    mx = jnp.max(l, axis=1, keepdims=True)
    io = lax.broadcasted_iota(jnp.int32, (N, 9), 1)
    pred = jnp.min(jnp.where(l == mx, io, 9), axis=1)
    preds_ref[...] = pred.reshape(1, 1, N)
    lse = mx[:, 0] + jnp.log(jnp.sum(jnp.exp(l - mx), axis=1))
    l_lab = jnp.sum(jnp.where(io == a_ref[...], l, 0.0), axis=1)
    part = jnp.sum(lse - l_lab).reshape(1, 1)

    @pl.when(s == 0)
    def _():
        loss_ref[...] = jnp.zeros((1, 1), jnp.float32)

    loss_ref[...] += part

    @pl.when(s == STEPS - 1)
    def _():
        loss_ref[...] = loss_ref[...] / float(STEPS * N)


def _readout(hs, a2, out_w, out_b2):
    return pl.pallas_call(
        _readout_body,
        grid=(STEPS,),
        in_specs=[
            pl.BlockSpec((1, N, H), lambda s: (s, 0, 0)),
            pl.BlockSpec((N, 1), lambda s: (0, 0)),
            _full((H, 9)),
            _full((1, 9)),
        ],
        out_specs=[
            pl.BlockSpec((1, 1, N), lambda s: (s, 0, 0)),
            pl.BlockSpec((1, 1), lambda s: (0, 0)),
        ],
        out_shape=[
            jax.ShapeDtypeStruct((STEPS, 1, N), jnp.int32),
            jax.ShapeDtypeStruct((1, 1), jnp.float32),
        ],
    )(hs, a2, out_w, out_b2)


NBUF = 2
HALVES = 2
EH = E_PAD // HALVES
HROWS = NROWS // HALVES
GH = EH // (NS * CHUNK)
NG = GH // NBUF

TRS = 624


def _sc_gather_body(half, p_hbm, q_hbm, src_hbm, dst_hbm, ea_hbm, eb_hbm,
                    table, idx_v, b0, b1, g0, g1, w0, w1):
    cid = lax.axis_index("c")
    sid = lax.axis_index("s")
    bufs = [b0, b1]
    gsem = [g0, g1]
    wsem = [w0, w1]
    hrow = half * HROWS

    def run(tab_hbm, idx_hbm, out_hbm):
        r0 = sid * TRS
        @pl.when(sid < NS - 1)
        def _():
            pltpu.sync_copy(tab_hbm.at[pl.ds(r0, TRS)], table.at[pl.ds(r0, TRS)])
        @pl.when(sid == NS - 1)
        def _():
            rem = N - (NS - 1) * TRS
            pltpu.sync_copy(tab_hbm.at[pl.ds((NS - 1) * TRS, rem)],
                            table.at[pl.ds((NS - 1) * TRS, rem)])
        pltpu.sync_copy(idx_hbm.at[pl.ds(hrow + sid * GH, GH)], idx_v)
        plsc.subcore_barrier()

        def fire_gather(b, g):
            j = g * NBUF + b
            pltpu.async_copy(table.at[idx_v.at[j]], bufs[b], gsem[b])

        def wait_gather(b):
            pltpu.make_async_copy(table.at[idx_v.at[0]], bufs[b], gsem[b]).wait()

        def fire_wb(b, g):
            j = g * NBUF + b
            off = (sid * GH + j) * CHUNK
            pltpu.async_copy(bufs[b], out_hbm.at[pl.ds(off, CHUNK)], wsem[b])

        def wait_wb(b):
            pltpu.make_async_copy(bufs[b], out_hbm.at[pl.ds(0, CHUNK)],
                                  wsem[b]).wait()

        for b in range(NBUF):
            fire_gather(b, 0)

        def body(g, carry):
            for b in range(NBUF):
                wait_gather(b)
                fire_wb(b, g)

            @pl.when(g + 1 < NG)
            def _():
                for b in range(NBUF):
                    wait_wb(b)
                    fire_gather(b, g + 1)

            return carry

        lax.fori_loop(0, NG, body, 0)
        for b in range(NBUF):
            wait_wb(b)

    @pl.when(cid == 0)
    def _():
        run(p_hbm, src_hbm, ea_hbm)

    @pl.when(cid == 1)
    def _():
        run(q_hbm, dst_hbm, eb_hbm)


def _sc_gather(p, q, src_r, dst_r, half):
    f32 = jnp.float32
    return pl.kernel(
        functools.partial(_sc_gather_body, half),
        [jax.ShapeDtypeStruct((EH, HP), f32)] * 2,
        mesh=plsc.VectorSubcoreMesh(core_axis_name="c", subcore_axis_name="s"),
        scratch_types=[
            pltpu.VMEM_SHARED((N, HP), f32),
            pltpu.VMEM((GH, CHUNK), jnp.int32),
        ] + [pltpu.VMEM((CHUNK, HP), f32)] * NBUF
          + [pltpu.SemaphoreType.DMA] * (2 * NBUF),
    )(p, q, src_r, dst_r)


SBUF = 2
SCPH = CPW // HALVES
SG = SCPH // SBUF


def _sc_scatter_body(half, e_hbm, dst_hbm, z_hbm, out_hbm, msh, idx_d,
                     b0, b1, r0, r1, a0, a1):
    cid = lax.axis_index("c")
    sid = lax.axis_index("s")
    wid = sid * NC + cid
    row0 = wid * SCPH
    pltpu.sync_copy(z_hbm.at[pl.ds(sid * RPS, RPS)], msh.at[pl.ds(sid * RPS, RPS)])
    pltpu.sync_copy(dst_hbm.at[half, wid], idx_d)
    plsc.subcore_barrier()

    bufs = [b0, b1]
    rsem = [r0, r1]
    asem = [a0, a1]

    def fire_read(b, g):
        j = g * SBUF + b
        off = (row0 + j) * CHUNK
        pltpu.async_copy(e_hbm.at[pl.ds(off, CHUNK)], bufs[b], rsem[b])

    def wait_read(b):
        pltpu.make_async_copy(e_hbm.at[pl.ds(row0 * CHUNK, CHUNK)], bufs[b],
                              rsem[b]).wait()

    def fire_add(b, g):
        j = g * SBUF + b
        pltpu.async_copy(bufs[b], msh.at[idx_d.at[j]], asem[b], add=True)

    def wait_add(b):
        pltpu.make_async_copy(bufs[b], msh.at[idx_d.at[0]], asem[b]).wait()

    for b in range(SBUF):
        fire_read(b, 0)

    def body(g, carry):
        for b in range(SBUF):
            wait_read(b)
            fire_add(b, g)

        @pl.when(g + 1 < SG)
        def _():
            for b in range(SBUF):
                wait_add(b)
                fire_read(b, g + 1)

        return carry

    lax.fori_loop(0, SG, body, 0)
    for b in range(SBUF):
        wait_add(b)
    plsc.subcore_barrier()
    pltpu.sync_copy(msh.at[pl.ds(sid * RPS, RPS)],
                    out_hbm.at[cid, pl.ds(sid * RPS, RPS)])


def _sc_scatter(e, dst_r, zrows, half):
    f32 = jnp.float32
    return pl.kernel(
        functools.partial(_sc_scatter_body, half),
        jax.ShapeDtypeStruct((NC, NM, HP), f32),
        mesh=plsc.VectorSubcoreMesh(core_axis_name="c", subcore_axis_name="s"),
        scratch_types=[
            pltpu.VMEM_SHARED((NM, HP), f32),
            pltpu.VMEM((SCPH, CHUNK), jnp.int32),
        ] + [pltpu.VMEM((CHUNK, HP), f32)] * SBUF
          + [pltpu.SemaphoreType.DMA] * (2 * SBUF),
    )(e, dst_r, zrows)


def kernel(q, a, edge_index, embed, in_W0, in_b0, in_W1, in_b1, in_W2, in_b2,
           in_W3, in_b3, msg_W0, msg_b0, msg_W1, msg_b1, msg_W2, msg_b2,
           msg_W3, msg_b3, W_ih, W_hh, out_W, out_b):
    f32 = jnp.float32
    i32 = jnp.int32
    q2 = q.astype(i32).reshape(N, 1)
    a2 = a.astype(i32).reshape(N, 1)
    ei = edge_index.astype(i32)
    pad = E_PAD - E
    src_r = jnp.concatenate([ei[0], jnp.zeros((pad,), i32)]).reshape(NROWS, CHUNK)
    dstg_r = jnp.concatenate([ei[1], jnp.zeros((pad,), i32)]).reshape(NROWS, CHUNK)
    dsts_r = jnp.concatenate([ei[1], jnp.full((pad,), N, i32)]).reshape(
        HALVES, NW, SCPH, CHUNK)
    zrows = jnp.zeros((NM, HP), f32)

    def padc(w):
        return jnp.pad(w, ((0, 0), (0, HP - w.shape[1])))

    def padr(w):
        return jnp.pad(w, ((0, HP - w.shape[0]), (0, 0)))

    iw = [in_W0, in_W1, in_W2, in_W3]
    ib = [b.reshape(1, H) for b in (in_b0, in_b1, in_b2, in_b3)]
    w0a = padc(msg_W0[:H])
    w0b = padc(msg_W0[H:])
    b0m = padc(msg_b0.reshape(1, H))
    w1p = padr(msg_W1)
    b1m = msg_b1.reshape(1, H)
    b2m = msg_b2.reshape(1, H)
    w3p = padc(msg_W3)
    b3p = padc(msg_b3.reshape(1, H))

    gw = []
    for k in range(4):
        gw.append(W_ih[k * H:(k + 1) * H, :H].T)
        gw.append(padr(W_ih[k * H:(k + 1) * H, H:].T))
        gw.append(W_hh[k * H:(k + 1) * H, :].T)

    x, p, qq = _input_mlp(q2, embed, iw, ib, w0a, w0b, b0m)
    h = x
    c = jnp.zeros((N, H), f32)
    hs_list = []
    for _ in range(STEPS):
        mps = []
        for hf in range(HALVES):
            ea, eb = _sc_gather(p, qq, src_r, dstg_r, hf)
            e = _edge_mlp(ea, eb, w1p, b1m, msg_W2, b2m, w3p, b3p)
            mps.append(_sc_scatter(e, dsts_r, zrows, hf))
        h, c, p, qq = _lstm(x, mps[0], mps[1], h, c, gw, w0a, w0b, b0m)
        hs_list.append(h)
    hs = jnp.stack(hs_list, 0)
    preds3, loss = _readout(hs, a2, out_W, out_b.reshape(1, 9))
    return preds3.reshape(STEPS, N), loss[0, 0]
